# baseline (speedup 1.0000x reference)
import math

import jax
import jax.numpy as jnp
from jax import lax
from jax.experimental import pallas as pl
from jax.experimental.pallas import tpu as pltpu
from jax.experimental.pallas import tpu_sc as plsc

_N = 10000
_E = 320000
_D = 128
_DH = _D // 2
_ALPHA = 0.1
_BETA = float(math.log(0.5 / 7.0 + 1.0))

_NC = 2
_NS = 16
_NW = _NC * _NS
_EPAD = 327680
_CHUNK = 512
_CROWS = _CHUNK // 128
_NPAD = 10240
_RPT = _NPAD // _NS


_TROWS = _EPAD // 128 // _NS
_SR0 = 160
_SRMAX = _SR0


def _seg_body(h0_hbm, h1_hbm, src_hbm, dst_hbm, attr_hbm, zero_hbm, out_hbm,
              src_v, dst_v, attr_v, rows_v, agg_sh, gsem, ssem):
    c = lax.axis_index("c")
    s = lax.axis_index("s")
    base_row = s * _SR0
    nchunk = _SR0 // _CROWS
    on_c0 = c == 0

    @pl.when(on_c0)
    def _():
        pltpu.sync_copy(src_hbm.at[pl.ds(base_row, _SRMAX)], src_v)

    def fire_gathers(ci, b, h_hbm):
        for j in range(_CROWS):
            pltpu.async_copy(h_hbm.at[src_v.at[ci * _CROWS + j]],
                             rows_v.at[b, pl.ds(j * 128, 128)], gsem)
        r0 = base_row + ci * _CROWS
        pltpu.async_copy(dst_hbm.at[pl.ds(r0, _CROWS)], dst_v.at[b], gsem)
        pltpu.async_copy(attr_hbm.at[pl.ds(r0, _CROWS)], attr_v.at[b], gsem)

    def wait_gathers(b, h_hbm):
        for j in range(_CROWS):
            pltpu.make_async_copy(h_hbm.at[src_v.at[j]],
                                  rows_v.at[b, pl.ds(j * 128, 128)],
                                  gsem).wait()
        pltpu.make_async_copy(dst_hbm.at[pl.ds(0, _CROWS)], dst_v.at[b],
                              gsem).wait()
        pltpu.make_async_copy(attr_hbm.at[pl.ds(0, _CROWS)], attr_v.at[b],
                              gsem).wait()

    def fire_scatters(ci, b):
        for j in range(_CROWS):
            pltpu.async_copy(rows_v.at[b, pl.ds(j * 128, 128)],
                             agg_sh.at[dst_v.at[b, j]],
                             ssem, add=True)

    def wait_scatters(b):
        for j in range(_CROWS):
            pltpu.make_async_copy(rows_v.at[b, pl.ds(j * 128, 128)],
                                  agg_sh.at[dst_v.at[0, j]], ssem).wait()

    for p, h_hbm in enumerate((h0_hbm, h1_hbm)):
      @pl.when(on_c0)
      def _(p=p, h_hbm=h_hbm):
        pltpu.sync_copy(zero_hbm.at[pl.ds(s * _RPT, _RPT)],
                        agg_sh.at[pl.ds(s * _RPT, _RPT)])
        plsc.subcore_barrier()

        fire_gathers(0, 0, h_hbm)

        def chunk_step(ci, carry, h_hbm=h_hbm):
            b = lax.rem(ci, 2)
            wait_gathers(b, h_hbm)
            @pl.when(ci > 0)
            def _():
                wait_scatters(1 - b)

            @pl.when(ci < nchunk - 1)
            def _():
                fire_gathers(ci + 1, 1 - b, h_hbm)
            for j in range(_CROWS):
                @plsc.parallel_loop(0, 128, unroll=8)
                def scale_edge(k, jj=j):
                    w = plsc.load_gather(
                        attr_v.at[b, jj],
                        [jnp.full((16,), k, jnp.int32)])
                    r = jj * 128 + k
                    for t in range(_DH // 16):
                        rows_v[b, r, pl.ds(16 * t, 16)] = (
                            rows_v[b, r, pl.ds(16 * t, 16)] * w)
            fire_scatters(ci, b)
            return carry

        lax.fori_loop(0, nchunk, chunk_step, 0)
        wait_scatters((nchunk - 1) % 2)
        plsc.subcore_barrier()
        pltpu.sync_copy(agg_sh.at[pl.ds(s * _RPT, _RPT)],
                        out_hbm.at[p, pl.ds(s * _RPT, _RPT)])
        plsc.subcore_barrier()


_seg_sum = pl.kernel(
    _seg_body,
    out_type=jax.ShapeDtypeStruct((2, _NPAD, _DH), jnp.float32),
    mesh=plsc.VectorSubcoreMesh(core_axis_name="c", subcore_axis_name="s"),
    compiler_params=pltpu.CompilerParams(
        needs_layout_passes=False, use_tc_tiling_on_sc=False),
    scratch_types=[
        pltpu.VMEM((_SRMAX, 128), jnp.int32),
        pltpu.VMEM((2, _CROWS, 128), jnp.int32),
        pltpu.VMEM((2, _CROWS, 128), jnp.float32),
        pltpu.VMEM((2, _CHUNK, _DH), jnp.float32),
        pltpu.VMEM_SHARED((_NPAD, _DH), jnp.float32),
        pltpu.SemaphoreType.DMA,
        pltpu.SemaphoreType.DMA,
    ],
)


_BR = 1000


def _entry_body(x_ref, w_ref, b_ref, h0_ref, h1_ref):
    acc = jnp.dot(x_ref[...], w_ref[...], preferred_element_type=jnp.float32)
    h = jnp.maximum(acc + b_ref[...], 0.0)
    h0_ref[...] = h[:, :_DH]
    h1_ref[...] = h[:, _DH:]


def _tc_entry(x, w, b):
    return pl.pallas_call(
        _entry_body,
        grid=(_N // _BR,),
        in_specs=[
            pl.BlockSpec((_BR, _D), lambda i: (i, 0)),
            pl.BlockSpec((_D, _D), lambda i: (0, 0)),
            pl.BlockSpec((1, _D), lambda i: (0, 0)),
        ],
        out_specs=[
            pl.BlockSpec((_BR, _DH), lambda i: (i, 0)),
            pl.BlockSpec((_BR, _DH), lambda i: (i, 0)),
        ],
        out_shape=[
            jax.ShapeDtypeStruct((_N, _DH), jnp.float32),
            jax.ShapeDtypeStruct((_N, _DH), jnp.float32),
        ],
    )(x, w, b)


def _conv_body(p_ref, x00_ref, x01_ref, w_ref, h0_ref, h1_ref):
    agg = p_ref[...]
    t0 = (1.0 - _ALPHA) * agg[0] + _ALPHA * x00_ref[...]
    t1 = (1.0 - _ALPHA) * agg[1] + _ALPHA * x01_ref[...]
    t = jnp.concatenate([t0, t1], axis=1)
    h = jnp.maximum(
        (1.0 - _BETA) * t
        + _BETA * jnp.dot(t, w_ref[...], preferred_element_type=jnp.float32),
        0.0)
    h0_ref[...] = h[:, :_DH]
    h1_ref[...] = h[:, _DH:]


def _tc_conv(parts, x00, x01, w):
    return pl.pallas_call(
        _conv_body,
        grid=(_N // _BR,),
        in_specs=[
            pl.BlockSpec((2, _BR, _DH), lambda i: (0, i, 0)),
            pl.BlockSpec((_BR, _DH), lambda i: (i, 0)),
            pl.BlockSpec((_BR, _DH), lambda i: (i, 0)),
            pl.BlockSpec((_D, _D), lambda i: (0, 0)),
        ],
        out_specs=[
            pl.BlockSpec((_BR, _DH), lambda i: (i, 0)),
            pl.BlockSpec((_BR, _DH), lambda i: (i, 0)),
        ],
        out_shape=[
            jax.ShapeDtypeStruct((_N, _DH), jnp.float32),
            jax.ShapeDtypeStruct((_N, _DH), jnp.float32),
        ],
    )(parts, x00, x01, w)


def _final_body(h0_ref, h1_ref, w_ref, b_ref, o_ref):
    h = jnp.concatenate([h0_ref[...], h1_ref[...]], axis=1)
    acc = jnp.dot(h, w_ref[...], preferred_element_type=jnp.float32)
    o_ref[...] = acc + b_ref[...]


def _tc_final(h0, h1, w, b):
    return pl.pallas_call(
        _final_body,
        grid=(_N // _BR,),
        in_specs=[
            pl.BlockSpec((_BR, _DH), lambda i: (i, 0)),
            pl.BlockSpec((_BR, _DH), lambda i: (i, 0)),
            pl.BlockSpec((_D, _D), lambda i: (0, 0)),
            pl.BlockSpec((1, _D), lambda i: (0, 0)),
        ],
        out_specs=pl.BlockSpec((_BR, _D), lambda i: (i, 0)),
        out_shape=jax.ShapeDtypeStruct((_N, _D), jnp.float32),
    )(h0, h1, w, b)


def kernel(x, edge_index, edge_attr, W0, b0, conv_ws, W1, b1):
    pad = _EPAD - _E
    src = jnp.concatenate(
        [edge_index[0], jnp.zeros((pad,), jnp.int32)]).reshape(_EPAD // 128, 128)
    dst = jnp.concatenate(
        [edge_index[1], jnp.zeros((pad,), jnp.int32)]).reshape(_EPAD // 128, 128)
    attr = jnp.concatenate(
        [edge_attr, jnp.zeros((pad,), jnp.float32)]).reshape(_EPAD // 128, 128)
    zero = jnp.zeros((_NPAD, _DH), jnp.float32)

    h0, h1 = _tc_entry(x, W0, b0.reshape(1, _D))
    x00, x01 = h0, h1
    for i in range(4):
        parts = _seg_sum(h0, h1, src, dst, attr, zero)
        h0, h1 = _tc_conv(parts, x00, x01, conv_ws[i])
    return _tc_final(h0, h1, W1, b1.reshape(1, _D))


# speedup vs baseline: 1.5698x; 1.5698x over previous
import math

import jax
import jax.numpy as jnp
from jax import lax
from jax.experimental import pallas as pl
from jax.experimental.pallas import tpu as pltpu
from jax.experimental.pallas import tpu_sc as plsc

_N = 10000
_E = 320000
_D = 128
_DH = _D // 2
_ALPHA = 0.1
_BETA = float(math.log(0.5 / 7.0 + 1.0))

_NC = 2
_NS = 16
_NW = _NC * _NS
_EPAD = 327680
_CHUNK = 512
_CROWS = _CHUNK // 128
_NPAD = 10240
_RPT = _NPAD // _NS


_TROWS = _EPAD // 128 // _NS
_SR0 = 152
_SR1 = _TROWS - _SR0
_SRMAX = _SR0


def _seg_body(h0_hbm, h1_hbm, src_hbm, dst_hbm, attr_hbm, zero_hbm, out_hbm,
              src_v, dst_v, attr_v, rows_v, agg_sh, gsem, ssem):
    c = lax.axis_index("c")
    s = lax.axis_index("s")
    base_row = jnp.where(c == 0, s * _SR0, _NS * _SR0 + s * _SR1)
    my_rows = jnp.where(c == 0, _SR0, _SR1)
    nchunk = my_rows // _CROWS

    pltpu.sync_copy(src_hbm.at[pl.ds(base_row, _SRMAX)], src_v)

    def fire_gathers(ci, b, h_hbm):
        for j in range(_CROWS):
            pltpu.async_copy(h_hbm.at[src_v.at[ci * _CROWS + j]],
                             rows_v.at[b, pl.ds(j * 128, 128)], gsem)
        r0 = base_row + ci * _CROWS
        pltpu.async_copy(dst_hbm.at[pl.ds(r0, _CROWS)], dst_v.at[b], gsem)
        pltpu.async_copy(attr_hbm.at[pl.ds(r0, _CROWS)], attr_v.at[b], gsem)

    def wait_gathers(b, h_hbm):
        for j in range(_CROWS):
            pltpu.make_async_copy(h_hbm.at[src_v.at[j]],
                                  rows_v.at[b, pl.ds(j * 128, 128)],
                                  gsem).wait()
        pltpu.make_async_copy(dst_hbm.at[pl.ds(0, _CROWS)], dst_v.at[b],
                              gsem).wait()
        pltpu.make_async_copy(attr_hbm.at[pl.ds(0, _CROWS)], attr_v.at[b],
                              gsem).wait()

    def fire_scatters(ci, b):
        for j in range(_CROWS):
            pltpu.async_copy(rows_v.at[b, pl.ds(j * 128, 128)],
                             agg_sh.at[dst_v.at[b, j]],
                             ssem, add=True)

    def wait_scatters(b):
        for j in range(_CROWS):
            pltpu.make_async_copy(rows_v.at[b, pl.ds(j * 128, 128)],
                                  agg_sh.at[dst_v.at[0, j]], ssem).wait()

    for p, h_hbm in enumerate((h0_hbm, h1_hbm)):
        pltpu.sync_copy(zero_hbm.at[pl.ds(s * _RPT, _RPT)],
                        agg_sh.at[pl.ds(s * _RPT, _RPT)])
        plsc.subcore_barrier()

        fire_gathers(0, 0, h_hbm)

        def chunk_step(ci, carry, h_hbm=h_hbm):
            b = lax.rem(ci, 2)
            wait_gathers(b, h_hbm)
            @pl.when(ci > 0)
            def _():
                wait_scatters(1 - b)

            @pl.when(ci < nchunk - 1)
            def _():
                fire_gathers(ci + 1, 1 - b, h_hbm)
            for j in range(_CROWS):
                @plsc.parallel_loop(0, 128, unroll=8)
                def scale_edge(k, jj=j):
                    w = plsc.load_gather(
                        attr_v.at[b, jj],
                        [jnp.full((16,), k, jnp.int32)])
                    r = jj * 128 + k
                    for t in range(_DH // 16):
                        rows_v[b, r, pl.ds(16 * t, 16)] = (
                            rows_v[b, r, pl.ds(16 * t, 16)] * w)
            fire_scatters(ci, b)
            return carry

        lax.fori_loop(0, nchunk, chunk_step, 0)
        wait_scatters(lax.rem(nchunk - 1, 2))
        plsc.subcore_barrier()
        pltpu.sync_copy(agg_sh.at[pl.ds(s * _RPT, _RPT)],
                        out_hbm.at[c, p, pl.ds(s * _RPT, _RPT)])
        plsc.subcore_barrier()


_seg_sum = pl.kernel(
    _seg_body,
    out_type=jax.ShapeDtypeStruct((_NC, 2, _NPAD, _DH), jnp.float32),
    mesh=plsc.VectorSubcoreMesh(core_axis_name="c", subcore_axis_name="s"),
    compiler_params=pltpu.CompilerParams(
        needs_layout_passes=False, use_tc_tiling_on_sc=False),
    scratch_types=[
        pltpu.VMEM((_SRMAX, 128), jnp.int32),
        pltpu.VMEM((2, _CROWS, 128), jnp.int32),
        pltpu.VMEM((2, _CROWS, 128), jnp.float32),
        pltpu.VMEM((2, _CHUNK, _DH), jnp.float32),
        pltpu.VMEM_SHARED((_NPAD, _DH), jnp.float32),
        pltpu.SemaphoreType.DMA,
        pltpu.SemaphoreType.DMA,
    ],
)


_BR = 1000


def _entry_body(x_ref, w_ref, b_ref, h0_ref, h1_ref):
    acc = jnp.dot(x_ref[...], w_ref[...], preferred_element_type=jnp.float32)
    h = jnp.maximum(acc + b_ref[...], 0.0)
    h0_ref[...] = h[:, :_DH]
    h1_ref[...] = h[:, _DH:]


def _tc_entry(x, w, b):
    return pl.pallas_call(
        _entry_body,
        grid=(_N // _BR,),
        in_specs=[
            pl.BlockSpec((_BR, _D), lambda i: (i, 0)),
            pl.BlockSpec((_D, _D), lambda i: (0, 0)),
            pl.BlockSpec((1, _D), lambda i: (0, 0)),
        ],
        out_specs=[
            pl.BlockSpec((_BR, _DH), lambda i: (i, 0)),
            pl.BlockSpec((_BR, _DH), lambda i: (i, 0)),
        ],
        out_shape=[
            jax.ShapeDtypeStruct((_N, _DH), jnp.float32),
            jax.ShapeDtypeStruct((_N, _DH), jnp.float32),
        ],
    )(x, w, b)


def _conv_body(p0_ref, p1_ref, x00_ref, x01_ref, w_ref, h0_ref, h1_ref):
    agg = p0_ref[0] + p1_ref[0]
    t0 = (1.0 - _ALPHA) * agg[0] + _ALPHA * x00_ref[...]
    t1 = (1.0 - _ALPHA) * agg[1] + _ALPHA * x01_ref[...]
    t = jnp.concatenate([t0, t1], axis=1)
    h = jnp.maximum(
        (1.0 - _BETA) * t
        + _BETA * jnp.dot(t, w_ref[...], preferred_element_type=jnp.float32),
        0.0)
    h0_ref[...] = h[:, :_DH]
    h1_ref[...] = h[:, _DH:]


def _tc_conv(parts, x00, x01, w):
    return pl.pallas_call(
        _conv_body,
        grid=(_N // _BR,),
        in_specs=[
            pl.BlockSpec((1, 2, _BR, _DH), lambda i: (0, 0, i, 0)),
            pl.BlockSpec((1, 2, _BR, _DH), lambda i: (1, 0, i, 0)),
            pl.BlockSpec((_BR, _DH), lambda i: (i, 0)),
            pl.BlockSpec((_BR, _DH), lambda i: (i, 0)),
            pl.BlockSpec((_D, _D), lambda i: (0, 0)),
        ],
        out_specs=[
            pl.BlockSpec((_BR, _DH), lambda i: (i, 0)),
            pl.BlockSpec((_BR, _DH), lambda i: (i, 0)),
        ],
        out_shape=[
            jax.ShapeDtypeStruct((_N, _DH), jnp.float32),
            jax.ShapeDtypeStruct((_N, _DH), jnp.float32),
        ],
    )(parts, parts, x00, x01, w)


def _final_body(h0_ref, h1_ref, w_ref, b_ref, o_ref):
    h = jnp.concatenate([h0_ref[...], h1_ref[...]], axis=1)
    acc = jnp.dot(h, w_ref[...], preferred_element_type=jnp.float32)
    o_ref[...] = acc + b_ref[...]


def _tc_final(h0, h1, w, b):
    return pl.pallas_call(
        _final_body,
        grid=(_N // _BR,),
        in_specs=[
            pl.BlockSpec((_BR, _DH), lambda i: (i, 0)),
            pl.BlockSpec((_BR, _DH), lambda i: (i, 0)),
            pl.BlockSpec((_D, _D), lambda i: (0, 0)),
            pl.BlockSpec((1, _D), lambda i: (0, 0)),
        ],
        out_specs=pl.BlockSpec((_BR, _D), lambda i: (i, 0)),
        out_shape=jax.ShapeDtypeStruct((_N, _D), jnp.float32),
    )(h0, h1, w, b)


def kernel(x, edge_index, edge_attr, W0, b0, conv_ws, W1, b1):
    pad = _EPAD - _E
    src = jnp.concatenate(
        [edge_index[0], jnp.zeros((pad,), jnp.int32)]).reshape(_EPAD // 128, 128)
    dst = jnp.concatenate(
        [edge_index[1], jnp.zeros((pad,), jnp.int32)]).reshape(_EPAD // 128, 128)
    attr = jnp.concatenate(
        [edge_attr, jnp.zeros((pad,), jnp.float32)]).reshape(_EPAD // 128, 128)
    zero = jnp.zeros((_NPAD, _DH), jnp.float32)

    h0, h1 = _tc_entry(x, W0, b0.reshape(1, _D))
    x00, x01 = h0, h1
    for i in range(4):
        parts = _seg_sum(h0, h1, src, dst, attr, zero)
        h0, h1 = _tc_conv(parts, x00, x01, conv_ws[i])
    return _tc_final(h0, h1, W1, b1.reshape(1, _D))
